# baseline (speedup 1.0000x reference)
import functools

import jax
import jax.numpy as jnp
from jax.experimental import pallas as pl
from jax.experimental.pallas import tpu as pltpu

K = 32
BLK = 256


def _bits_to_bf16(bits_i32):
    return jax.lax.bitcast_convert_type(bits_i32.astype(jnp.int16), jnp.bfloat16)


def _bf16_to_bits(x_bf16):
    return jax.lax.bitcast_convert_type(x_bf16, jnp.int16).astype(jnp.int32)


def _fold(s, op, width):
    while s.shape[1] > width:
        h = s.shape[1] // 2
        s = op(s[:, :h], s[:, h:])
    return s


def _body(a_ref, b2_ref, out_ref, dbf_ref):
    i = pl.program_id(0)

    a = a_ref[...]
    b2 = b2_ref[...]

    g = jax.lax.dot_general(a, b2, (((1,), (0,)), ((), ())),
                            preferred_element_type=jnp.float32)
    dbf_ref[...] = jnp.maximum(g, 0.0).astype(jnp.bfloat16)

    x0 = dbf_ref[...]
    rmax = _fold(x0, jnp.maximum, 16)
    rmax = jnp.max(rmax.astype(jnp.float32), axis=1, keepdims=True)

    win = dbf_ref[:, pl.ds(i * BLK, BLK)]
    rl = jax.lax.broadcasted_iota(jnp.int32, (BLK, BLK), 0)
    cl = jax.lax.broadcasted_iota(jnp.int32, (BLK, BLK), 1)
    dbf_ref[:, pl.ds(i * BLK, BLK)] = jnp.where(rl == cl, jnp.inf, win)

    x = dbf_ref[...]
    rmin = _fold(x, jnp.minimum, 16)
    rmin = jnp.min(rmin.astype(jnp.float32), axis=1, keepdims=True)
    lo0 = _bf16_to_bits(rmin.astype(jnp.bfloat16)) - 1
    hi0 = _bf16_to_bits(rmax.astype(jnp.bfloat16))

    need = jnp.float32(K + 1)

    def cond(carry):
        lo, hi = carry
        return jnp.max(hi - lo) > 1

    def bis(carry):
        lo, hi = carry
        mid = (lo + hi) >> 1
        thr = _bits_to_bf16(mid)
        d = dbf_ref[...]
        s = jnp.where(d <= thr, jnp.bfloat16(1.0), jnp.bfloat16(0.0))
        s = _fold(s, jnp.add, 16)
        cnt = jnp.sum(s.astype(jnp.float32), axis=1, keepdims=True)
        ge = cnt >= need
        return jnp.where(ge, lo, mid), jnp.where(ge, mid, hi)

    _, hi = jax.lax.while_loop(cond, bis, (lo0, hi0))
    out_ref[...] = jnp.sqrt(_bits_to_bf16(hi).astype(jnp.float32))


@functools.partial(jax.jit, static_argnames=())
def kernel(images):
    n, d = images.shape
    a16 = images.astype(jnp.bfloat16)
    bf = a16.astype(jnp.float32)
    sq16 = jnp.sum(bf * bf, axis=1).astype(jnp.bfloat16)
    ones = jnp.ones((n, 1), jnp.bfloat16)
    zeros3 = jnp.zeros((n, 6), jnp.bfloat16)
    a_aug = jnp.concatenate([a16, sq16[:, None], ones, zeros3], axis=1)
    b2_aug = jnp.concatenate(
        [(-2.0 * bf).astype(jnp.bfloat16), ones, sq16[:, None], zeros3],
        axis=1).T
    daug = d + 8

    out = pl.pallas_call(
        _body,
        grid=(n // BLK,),
        in_specs=[
            pl.BlockSpec((BLK, daug), lambda i: (i, 0)),
            pl.BlockSpec((daug, n), lambda i: (0, 0)),
        ],
        out_specs=pl.BlockSpec((BLK, 1), lambda i: (i, 0)),
        out_shape=jax.ShapeDtypeStruct((n, 1), jnp.float32),
        scratch_shapes=[pltpu.VMEM((BLK, n), jnp.bfloat16)],
    )(a_aug, b2_aug)
    return out[:, 0]


# speedup vs baseline: 1.7005x; 1.7005x over previous
import functools

import jax
import jax.numpy as jnp
from jax.experimental import pallas as pl
from jax.experimental.pallas import tpu as pltpu

K = 32
BLK = 512


def _bits_to_bf16(bits_i32):
    return jax.lax.bitcast_convert_type(bits_i32.astype(jnp.int16), jnp.bfloat16)


def _bf16_to_bits(x_bf16):
    return jax.lax.bitcast_convert_type(x_bf16, jnp.int16).astype(jnp.int32)


def _fold(s, op, width):
    while s.shape[1] > width:
        h = s.shape[1] // 2
        s = op(s[:, :h], s[:, h:])
    return s


def _body(a_ref, b2_ref, sq_ref, rsq_ref, out_ref, dbf_ref):
    i = pl.program_id(0)

    a = a_ref[...]
    b2 = b2_ref[...]

    row_sq = rsq_ref[...]
    col_sq = sq_ref[...]

    g2 = jax.lax.dot_general(a, b2, (((1,), (0,)), ((), ())),
                             preferred_element_type=jnp.float32)
    dsq = jnp.maximum((row_sq + col_sq) + g2, 0.0)
    dbf_ref[...] = dsq.astype(jnp.bfloat16)

    x0 = dbf_ref[...]
    rmax = _fold(x0, jnp.maximum, 16)
    rmax = jnp.max(rmax.astype(jnp.float32), axis=1, keepdims=True)

    win = dbf_ref[:, pl.ds(i * BLK, BLK)]
    rl = jax.lax.broadcasted_iota(jnp.int32, (BLK, BLK), 0)
    cl = jax.lax.broadcasted_iota(jnp.int32, (BLK, BLK), 1)
    dbf_ref[:, pl.ds(i * BLK, BLK)] = jnp.where(rl == cl, jnp.inf, win)

    x = dbf_ref[...]
    rmin = _fold(x, jnp.minimum, 16)
    rmin = jnp.min(rmin.astype(jnp.float32), axis=1, keepdims=True)
    lo0 = _bf16_to_bits(rmin.astype(jnp.bfloat16)) - 1
    hi0 = _bf16_to_bits(rmax.astype(jnp.bfloat16))

    need = jnp.float32(K + 1)
    span0 = jnp.max(hi0 - lo0)

    def cond(carry):
        _, _, span = carry
        return span > 1

    def bis(carry):
        lo, hi, span = carry
        mid = (lo + hi) >> 1
        thr = _bits_to_bf16(mid)
        d = dbf_ref[...]
        s = jnp.where(d <= thr, jnp.bfloat16(1.0), jnp.bfloat16(0.0))
        s = _fold(s, jnp.add, 16)
        cnt = jnp.sum(s.astype(jnp.float32), axis=1, keepdims=True)
        ge = cnt >= need
        return (jnp.where(ge, lo, mid), jnp.where(ge, mid, hi),
                (span + 1) >> 1)

    _, hi, _ = jax.lax.while_loop(cond, bis, (lo0, hi0, span0))
    out_ref[...] = jnp.sqrt(_bits_to_bf16(hi).astype(jnp.float32))


@functools.partial(jax.jit, static_argnames=())
def kernel(images):
    n, d = images.shape
    a16 = images.astype(jnp.bfloat16)
    b2 = (-2.0 * a16.astype(jnp.float32)).astype(jnp.bfloat16).T
    bf = a16.astype(jnp.float32)
    sq = jnp.sum(bf * bf, axis=1)
    col_sq = sq[None, :]
    row_sq = sq[:, None]

    out = pl.pallas_call(
        _body,
        grid=(n // BLK,),
        in_specs=[
            pl.BlockSpec((BLK, d), lambda i: (i, 0)),
            pl.BlockSpec((d, n), lambda i: (0, 0)),
            pl.BlockSpec((1, n), lambda i: (0, 0)),
            pl.BlockSpec((BLK, 1), lambda i: (i, 0)),
        ],
        out_specs=pl.BlockSpec((BLK, 1), lambda i: (i, 0)),
        out_shape=jax.ShapeDtypeStruct((n, 1), jnp.float32),
        scratch_shapes=[pltpu.VMEM((BLK, n), jnp.bfloat16)],
    )(a16, b2, col_sq, row_sq)
    return out[:, 0]


# speedup vs baseline: 1.7013x; 1.0005x over previous
import functools

import jax
import jax.numpy as jnp
from jax.experimental import pallas as pl
from jax.experimental.pallas import tpu as pltpu

K = 32
BLK = 512


def _bits_to_bf16(bits_i32):
    return jax.lax.bitcast_convert_type(bits_i32.astype(jnp.int16), jnp.bfloat16)


def _bf16_to_bits(x_bf16):
    return jax.lax.bitcast_convert_type(x_bf16, jnp.int16).astype(jnp.int32)


def _fold(s, op, width):
    while s.shape[1] > width:
        h = s.shape[1] // 2
        s = op(s[:, :h], s[:, h:])
    return s


def _body(a_ref, b2_ref, sq_ref, rsq_ref, out_ref, dbf_ref):
    i = pl.program_id(0)

    a = a_ref[...]
    b2 = b2_ref[...]

    row_sq = rsq_ref[...]
    col_sq = sq_ref[...]

    g2 = jax.lax.dot_general(a, b2, (((1,), (0,)), ((), ())),
                             preferred_element_type=jnp.float32)
    g2bf = g2.astype(jnp.bfloat16)
    dbf_ref[...] = jnp.maximum((row_sq + col_sq) + g2bf, jnp.bfloat16(0.0))

    x0 = dbf_ref[...]
    rmax = _fold(x0, jnp.maximum, 16)
    rmax = jnp.max(rmax.astype(jnp.float32), axis=1, keepdims=True)

    win = dbf_ref[:, pl.ds(i * BLK, BLK)]
    rl = jax.lax.broadcasted_iota(jnp.int32, (BLK, BLK), 0)
    cl = jax.lax.broadcasted_iota(jnp.int32, (BLK, BLK), 1)
    dbf_ref[:, pl.ds(i * BLK, BLK)] = jnp.where(rl == cl, jnp.inf, win)

    x = dbf_ref[...]
    rmin = _fold(x, jnp.minimum, 16)
    rmin = jnp.min(rmin.astype(jnp.float32), axis=1, keepdims=True)
    lo0 = _bf16_to_bits(rmin.astype(jnp.bfloat16)) - 1
    hi0 = _bf16_to_bits(rmax.astype(jnp.bfloat16))

    need = jnp.float32(K + 1)
    span0 = jnp.max(hi0 - lo0)

    def cond(carry):
        _, _, span = carry
        return span > 1

    def bis(carry):
        lo, hi, span = carry
        mid = (lo + hi) >> 1
        thr = _bits_to_bf16(mid)
        d = dbf_ref[...]
        s = jnp.where(d <= thr, jnp.bfloat16(1.0), jnp.bfloat16(0.0))
        s = _fold(s, jnp.add, 16)
        cnt = jnp.sum(s.astype(jnp.float32), axis=1, keepdims=True)
        ge = cnt >= need
        return (jnp.where(ge, lo, mid), jnp.where(ge, mid, hi),
                (span + 1) >> 1)

    _, hi, _ = jax.lax.while_loop(cond, bis, (lo0, hi0, span0))
    out_ref[...] = jnp.sqrt(_bits_to_bf16(hi).astype(jnp.float32))


@functools.partial(jax.jit, static_argnames=())
def kernel(images):
    n, d = images.shape
    a16 = images.astype(jnp.bfloat16)
    b2 = (-2.0 * a16.astype(jnp.float32)).astype(jnp.bfloat16).T
    bf = a16.astype(jnp.float32)
    sq = jnp.sum(bf * bf, axis=1).astype(jnp.bfloat16)
    col_sq = sq[None, :]
    row_sq = sq[:, None]

    out = pl.pallas_call(
        _body,
        grid=(n // BLK,),
        in_specs=[
            pl.BlockSpec((BLK, d), lambda i: (i, 0)),
            pl.BlockSpec((d, n), lambda i: (0, 0)),
            pl.BlockSpec((1, n), lambda i: (0, 0)),
            pl.BlockSpec((BLK, 1), lambda i: (i, 0)),
        ],
        out_specs=pl.BlockSpec((BLK, 1), lambda i: (i, 0)),
        out_shape=jax.ShapeDtypeStruct((n, 1), jnp.float32),
        scratch_shapes=[pltpu.VMEM((BLK, n), jnp.bfloat16)],
    )(a16, b2, col_sq, row_sq)
    return out[:, 0]


# speedup vs baseline: 1.7709x; 1.0409x over previous
import functools

import jax
import jax.numpy as jnp
from jax.experimental import pallas as pl
from jax.experimental.pallas import tpu as pltpu

K = 32
BLK = 512


def _bits_to_bf16(bits_i32):
    return jax.lax.bitcast_convert_type(bits_i32.astype(jnp.int16), jnp.bfloat16)


def _bf16_to_bits(x_bf16):
    return jax.lax.bitcast_convert_type(x_bf16, jnp.int16).astype(jnp.int32)


def _fold(s, op, width):
    while s.shape[1] > width:
        h = s.shape[1] // 2
        s = op(s[:, :h], s[:, h:])
    return s


def _body(a_ref, b2_ref, sq_ref, rsq_ref, out_ref, dbf_ref):
    i = pl.program_id(0)

    a = a_ref[...]
    b2 = b2_ref[...]

    row_sq = rsq_ref[...]
    col_sq = sq_ref[...]

    g2 = jax.lax.dot_general(a, b2, (((1,), (0,)), ((), ())),
                             preferred_element_type=jnp.float32)
    g2bf = g2.astype(jnp.bfloat16)
    dbf_ref[...] = jnp.maximum((row_sq + col_sq) + g2bf, jnp.bfloat16(0.0))

    x0 = dbf_ref[...]
    rmax = _fold(x0, jnp.maximum, 16)
    rmax = jnp.max(rmax.astype(jnp.float32), axis=1, keepdims=True)

    win = dbf_ref[:, pl.ds(i * BLK, BLK)]
    rl = jax.lax.broadcasted_iota(jnp.int32, (BLK, BLK), 0)
    cl = jax.lax.broadcasted_iota(jnp.int32, (BLK, BLK), 1)
    dbf_ref[:, pl.ds(i * BLK, BLK)] = jnp.where(rl == cl, jnp.inf, win)

    x = dbf_ref[...]
    rmin = _fold(x, jnp.minimum, 16)
    rmin = jnp.min(rmin.astype(jnp.float32), axis=1, keepdims=True)
    lo0 = _bf16_to_bits(rmin.astype(jnp.bfloat16)) - 1
    hi0 = _bf16_to_bits(rmax.astype(jnp.bfloat16))

    need = jnp.float32(K + 1)
    span0 = jnp.max(hi0 - lo0)

    def cond(carry):
        _, _, span = carry
        return span > 1

    def bis(carry):
        lo, hi, span = carry
        mid = (lo + hi) >> 1
        thr = _bits_to_bf16(mid)
        d = dbf_ref[...]
        s = jnp.where(d <= thr, jnp.bfloat16(1.0), jnp.bfloat16(0.0))
        s = _fold(s, jnp.add, 16)
        cnt = jnp.sum(s.astype(jnp.float32), axis=1, keepdims=True)
        ge = cnt >= need
        return (jnp.where(ge, lo, mid), jnp.where(ge, mid, hi),
                (span + 1) >> 1)

    _, hi, _ = jax.lax.while_loop(cond, bis, (lo0, hi0, span0))
    out_ref[...] = jnp.sqrt(_bits_to_bf16(hi).astype(jnp.float32))


@functools.partial(jax.jit, static_argnames=())
def kernel(images):
    n, d = images.shape
    a8 = images.astype(jnp.float8_e4m3fn)
    bf = a8.astype(jnp.float32)
    b2 = (-2.0 * bf).astype(jnp.float8_e4m3fn).T
    sq = jnp.sum(bf * bf, axis=1).astype(jnp.bfloat16)
    col_sq = sq[None, :]
    row_sq = sq[:, None]

    out = pl.pallas_call(
        _body,
        grid=(n // BLK,),
        in_specs=[
            pl.BlockSpec((BLK, d), lambda i: (i, 0)),
            pl.BlockSpec((d, n), lambda i: (0, 0)),
            pl.BlockSpec((1, n), lambda i: (0, 0)),
            pl.BlockSpec((BLK, 1), lambda i: (i, 0)),
        ],
        out_specs=pl.BlockSpec((BLK, 1), lambda i: (i, 0)),
        out_shape=jax.ShapeDtypeStruct((n, 1), jnp.float32),
        scratch_shapes=[pltpu.VMEM((BLK, n), jnp.bfloat16)],
    )(a8, b2, col_sq, row_sq)
    return out[:, 0]


# speedup vs baseline: 1.7765x; 1.0032x over previous
import functools

import jax
import jax.numpy as jnp
from jax.experimental import pallas as pl
from jax.experimental.pallas import tpu as pltpu

K = 32
BLK = 1024


def _bits_to_bf16(bits_i32):
    return jax.lax.bitcast_convert_type(bits_i32.astype(jnp.int16), jnp.bfloat16)


def _bf16_to_bits(x_bf16):
    return jax.lax.bitcast_convert_type(x_bf16, jnp.int16).astype(jnp.int32)


def _fold(s, op, width):
    while s.shape[1] > width:
        h = s.shape[1] // 2
        s = op(s[:, :h], s[:, h:])
    return s


def _body(a_ref, b2_ref, sq_ref, rsq_ref, out_ref, dbf_ref):
    i = pl.program_id(0)

    a = a_ref[...]
    b2 = b2_ref[...]

    row_sq = rsq_ref[...]
    col_sq = sq_ref[...]

    g2 = jax.lax.dot_general(a, b2, (((1,), (0,)), ((), ())),
                             preferred_element_type=jnp.float32)
    g2bf = g2.astype(jnp.bfloat16)
    dbf_ref[...] = jnp.maximum((row_sq + col_sq) + g2bf, jnp.bfloat16(0.0))

    x0 = dbf_ref[...]
    rmax = _fold(x0, jnp.maximum, 16)
    rmax = jnp.max(rmax.astype(jnp.float32), axis=1, keepdims=True)

    win = dbf_ref[:, pl.ds(i * BLK, BLK)]
    rl = jax.lax.broadcasted_iota(jnp.int32, (BLK, BLK), 0)
    cl = jax.lax.broadcasted_iota(jnp.int32, (BLK, BLK), 1)
    dbf_ref[:, pl.ds(i * BLK, BLK)] = jnp.where(rl == cl, jnp.inf, win)

    x = dbf_ref[...]
    rmin = _fold(x, jnp.minimum, 16)
    rmin = jnp.min(rmin.astype(jnp.float32), axis=1, keepdims=True)
    lo0 = _bf16_to_bits(rmin.astype(jnp.bfloat16)) - 1
    hi0 = _bf16_to_bits(rmax.astype(jnp.bfloat16))

    need = jnp.float32(K + 1)
    span0 = jnp.max(hi0 - lo0)

    def cond(carry):
        _, _, span = carry
        return span > 1

    def bis(carry):
        lo, hi, span = carry
        mid = (lo + hi) >> 1
        thr = _bits_to_bf16(mid)
        d = dbf_ref[...]
        s = jnp.where(d <= thr, jnp.bfloat16(1.0), jnp.bfloat16(0.0))
        s = _fold(s, jnp.add, 16)
        cnt = jnp.sum(s.astype(jnp.float32), axis=1, keepdims=True)
        ge = cnt >= need
        return (jnp.where(ge, lo, mid), jnp.where(ge, mid, hi),
                (span + 1) >> 1)

    _, hi, _ = jax.lax.while_loop(cond, bis, (lo0, hi0, span0))
    out_ref[...] = jnp.sqrt(_bits_to_bf16(hi).astype(jnp.float32))


@functools.partial(jax.jit, static_argnames=())
def kernel(images):
    n, d = images.shape
    a8 = images.astype(jnp.float8_e4m3fn)
    bf = a8.astype(jnp.float32)
    b2 = (-2.0 * bf).astype(jnp.float8_e4m3fn).T
    sq = jnp.sum(bf * bf, axis=1).astype(jnp.bfloat16)
    col_sq = sq[None, :]
    row_sq = sq[:, None]

    out = pl.pallas_call(
        _body,
        grid=(n // BLK,),
        in_specs=[
            pl.BlockSpec((BLK, d), lambda i: (i, 0)),
            pl.BlockSpec((d, n), lambda i: (0, 0)),
            pl.BlockSpec((1, n), lambda i: (0, 0)),
            pl.BlockSpec((BLK, 1), lambda i: (i, 0)),
        ],
        out_specs=pl.BlockSpec((BLK, 1), lambda i: (i, 0)),
        out_shape=jax.ShapeDtypeStruct((n, 1), jnp.float32),
        scratch_shapes=[pltpu.VMEM((BLK, n), jnp.bfloat16)],
    )(a8, b2, col_sq, row_sq)
    return out[:, 0]
